# baseline (speedup 1.0000x reference)
import jax
import jax.numpy as jnp
from jax.experimental import pallas as pl

N = 10000
E = 320000
F_IN = 128
H = 4
C = 32


def _epilogue_body(out_ref, mu_ref, inv_ref, gamma_ref, beta_ref, x_ref, y_ref):
    o = out_ref[...]
    y = (o - mu_ref[...]) * inv_ref[...] * gamma_ref[...] + beta_ref[...]
    y_ref[...] = jnp.maximum(y, 0.0) + x_ref[...]


def kernel(x, edge_index, W, att_src, att_dst, bias, gamma, beta):
    loop = jnp.arange(N, dtype=edge_index.dtype)
    src = jnp.concatenate([edge_index[0], loop])
    dst = jnp.concatenate([edge_index[1], loop])
    h = (x @ W).reshape(N, H, C)
    a_src = (h * att_src[None, :, :]).sum(-1)
    a_dst = (h * att_dst[None, :, :]).sum(-1)
    e = a_src[src] + a_dst[dst]
    e = jax.nn.leaky_relu(e, negative_slope=0.2)
    emax = jax.ops.segment_max(e, dst, num_segments=N)
    ex = jnp.exp(e - emax[dst])
    denom = jax.ops.segment_sum(ex, dst, num_segments=N)
    alpha = ex / (denom[dst] + 1e-16)
    msg = h[src] * alpha[:, :, None]
    out = jax.ops.segment_sum(msg, dst, num_segments=N)
    out = out.reshape(N, H * C) + bias
    mu = out.mean(axis=0)
    var = out.var(axis=0)
    inv = 1.0 / jnp.sqrt(var + 1e-5)
    R = 2000
    y = pl.pallas_call(
        _epilogue_body,
        out_shape=jax.ShapeDtypeStruct((N, H * C), jnp.float32),
        grid=(N // R,),
        in_specs=[
            pl.BlockSpec((R, H * C), lambda i: (i, 0)),
            pl.BlockSpec((H * C,), lambda i: (0,)),
            pl.BlockSpec((H * C,), lambda i: (0,)),
            pl.BlockSpec((H * C,), lambda i: (0,)),
            pl.BlockSpec((H * C,), lambda i: (0,)),
            pl.BlockSpec((R, H * C), lambda i: (i, 0)),
        ],
        out_specs=pl.BlockSpec((R, H * C), lambda i: (i, 0)),
    )(out, mu, inv, gamma, beta, x)
    return y


# speedup vs baseline: 18.2202x; 18.2202x over previous
import dataclasses
import functools

import jax
import jax.numpy as jnp
from jax import lax
from jax.experimental import pallas as pl
from jax.experimental.pallas import tpu as pltpu
from jax.experimental.pallas import tpu_sc as plsc

N = 10000
E = 320000
F_IN = 128
H = 4
C = 32
HC = H * C
AW = HC + 16

NTILES = 32
EPW = E // NTILES
CHUNK = 80
NCHUNK = EPW // CHUNK
NPAD = 10240
RPT = NPAD // 16


_R1 = 1000


def _pre_body(x_ref, w_ref, as_ref, ad_ref, h_ref, asrc_ref, adst_ref,
              k_ref, mx_ref):
    i = pl.program_id(0)
    xb = x_ref[...]
    hb = jnp.dot(xb, w_ref[...], preferred_element_type=jnp.float32)
    h_ref[...] = hb
    a_s = jnp.dot(hb, as_ref[...], preferred_element_type=jnp.float32)
    a_d = jnp.dot(hb, ad_ref[...], preferred_element_type=jnp.float32)
    asrc_ref[...] = a_s
    adst_ref[...] = a_d
    prev_s = jnp.where(i == 0, -jnp.inf, mx_ref[0])
    prev_d = jnp.where(i == 0, -jnp.inf, mx_ref[1])
    ms = jnp.maximum(prev_s, jnp.max(a_s))
    md = jnp.maximum(prev_d, jnp.max(a_d))
    mx_ref[0] = ms
    mx_ref[1] = md
    m = ms + md
    k = jnp.maximum(m, 0.2 * m)
    k_ref[...] = jnp.full((8, HC), k, jnp.float32)


def _tc_pre(x, W, A_src, A_dst):
    return pl.pallas_call(
        _pre_body,
        grid=(N // _R1,),
        in_specs=[
            pl.BlockSpec((_R1, F_IN), lambda i: (i, 0)),
            pl.BlockSpec((F_IN, HC), lambda i: (0, 0)),
            pl.BlockSpec((HC, H), lambda i: (0, 0)),
            pl.BlockSpec((HC, H), lambda i: (0, 0)),
        ],
        out_specs=[
            pl.BlockSpec((_R1, HC), lambda i: (i, 0)),
            pl.BlockSpec((_R1, H), lambda i: (i, 0)),
            pl.BlockSpec((_R1, H), lambda i: (i, 0)),
            pl.BlockSpec((8, HC), lambda i: (0, 0)),
        ],
        out_shape=[
            jax.ShapeDtypeStruct((N, HC), jnp.float32),
            jax.ShapeDtypeStruct((N, H), jnp.float32),
            jax.ShapeDtypeStruct((N, H), jnp.float32),
            jax.ShapeDtypeStruct((8, HC), jnp.float32),
        ],
        scratch_shapes=[pltpu.SMEM((2,), jnp.float32)],
    )(x, W, A_src, A_dst)


DR = NPAD // 32

_SC_MESH = dict(core_axis_name="c", subcore_axis_name="s")


def _sc_cp():
    cp = pltpu.CompilerParams()
    if "needs_layout_passes" in pltpu.CompilerParams.__dataclass_fields__:
        cp = dataclasses.replace(cp, needs_layout_passes=False)
    return cp


def _sc_logits_body(asrc_hbm, adst_hbm, src_hbm, dst_hbm, k_hbm, t_hbm,
                    asrc_v, adst_v, kv, srci, dsti, tbuf, sem):
    cid = lax.axis_index("c")
    sid = lax.axis_index("s")
    iota16 = lax.iota(jnp.int32, 16)

    pltpu.sync_copy(asrc_hbm, asrc_v)
    pltpu.sync_copy(adst_hbm, adst_v)
    pltpu.sync_copy(k_hbm, kv)
    kvec = kv[...]
    base = (cid * 16 + sid) * EPW

    @pl.loop(0, NCHUNK)
    def _chunk(cn):
        off = base + cn * CHUNK
        pltpu.sync_copy(src_hbm.at[pl.ds(off, CHUNK)], srci)
        pltpu.sync_copy(dst_hbm.at[pl.ds(off, CHUNK)], dsti)

        @pl.loop(0, CHUNK // 16)
        def _grp(g):
            s16 = srci[pl.ds(g * 16, 16)]
            d16 = dsti[pl.ds(g * 16, 16)]
            rows4 = (g * 16 + iota16) * H
            s4 = s16 * H
            d4 = d16 * H
            for hh in range(H):
                a_s = plsc.load_gather(asrc_v, [s4 + hh])
                a_d = plsc.load_gather(adst_v, [d4 + hh])
                e = a_s + a_d
                e = jnp.maximum(e, 0.2 * e)
                t = jnp.exp(e - kvec)
                plsc.store_scatter(tbuf, [rows4 + hh], t)

        pltpu.sync_copy(tbuf, t_hbm.at[pl.ds(off * H, CHUNK * H)])


def _sc_logits(asrc_flat, adst_flat, src, dst, k16):
    kfn = pl.kernel(
        _sc_logits_body,
        mesh=plsc.VectorSubcoreMesh(**_SC_MESH),
        compiler_params=_sc_cp(),
        out_type=jax.ShapeDtypeStruct((E * H,), jnp.float32),
        scratch_types=[
            pltpu.VMEM((N * H,), jnp.float32),
            pltpu.VMEM((N * H,), jnp.float32),
            pltpu.VMEM((16,), jnp.float32),
            pltpu.VMEM((CHUNK,), jnp.int32),
            pltpu.VMEM((CHUNK,), jnp.int32),
            pltpu.VMEM((CHUNK * H,), jnp.float32),
            pltpu.SemaphoreType.DMA,
        ],
    )
    return kfn(asrc_flat, adst_flat, src, dst, k16)


def _sc_body(h_hbm, src_hbm, dst_hbm, t_hbm, z_hbm,
             out_hbm, outd_hbm, srci, dsti, dsti2, tbuf,
             gin, tden, acc, accd, sem_g, sem_s):
    cid = lax.axis_index("c")
    sid = lax.axis_index("s")
    iota16 = lax.iota(jnp.int32, 16)

    pltpu.sync_copy(z_hbm, tden)

    @pl.loop(0, RPT // CHUNK)
    def _za(j):
        pltpu.sync_copy(z_hbm, acc.at[pl.ds(sid * RPT + j * CHUNK, CHUNK)])

    @pl.when(sid < DR // 32)
    def _zd():
        pltpu.sync_copy(z_hbm.at[pl.ds(0, 32)], accd.at[pl.ds(sid * 32, 32)])

    plsc.subcore_barrier()

    zero16 = jnp.zeros((16,), jnp.float32)
    base = (cid * 16 + sid) * EPW

    @pl.loop(0, NCHUNK)
    def _chunk(cn):
        off = base + cn * CHUNK
        pltpu.sync_copy(src_hbm.at[pl.ds(off, CHUNK)], srci)
        pltpu.sync_copy(dst_hbm.at[pl.ds(off, CHUNK)], dsti)
        pltpu.sync_copy(t_hbm.at[pl.ds(off * H, CHUNK * H)], tbuf)
        pltpu.async_copy(h_hbm.at[srci], gin, sem_g).wait()

        @pl.loop(0, CHUNK // 16)
        def _grp(g):
            d16 = dsti[pl.ds(g * 16, 16)]
            rows = g * 16 + iota16
            rows4 = rows * H
            cdt = (d16 & 31) * H
            dsti2[pl.ds(g * 16, 16)] = lax.shift_right_logical(d16, 5)
            for hh in range(H):
                t = plsc.load_gather(tbuf, [rows4 + hh])
                plsc.store_scatter(tden, [rows, cdt + hh], t)
                for cc in range(hh * C, (hh + 1) * C):
                    colv = jnp.full((16,), cc, jnp.int32)
                    v = plsc.load_gather(gin, [rows, colv])
                    plsc.store_scatter(gin, [rows, colv], v * t)

        pltpu.sync_copy(gin, acc.at[dsti], add=True)
        pltpu.sync_copy(tden, accd.at[dsti2], add=True)

        @pl.loop(0, CHUNK // 16)
        def _rz(g):
            d16 = dsti[pl.ds(g * 16, 16)]
            rows = g * 16 + iota16
            cdt = (d16 & 31) * H
            for hh in range(H):
                plsc.store_scatter(tden, [rows, cdt + hh], zero16)

    plsc.subcore_barrier()
    pltpu.sync_copy(acc.at[pl.ds(sid * RPT, RPT)],
                    out_hbm.at[cid, pl.ds(sid * RPT, RPT)])

    @pl.when(sid == 0)
    def _dd():
        pltpu.sync_copy(accd, outd_hbm.at[cid])


def _sc_edges(h, src, dst, tflat):
    kfn = pl.kernel(
        _sc_body,
        mesh=plsc.VectorSubcoreMesh(**_SC_MESH),
        compiler_params=_sc_cp(),
        out_type=[
            jax.ShapeDtypeStruct((2, NPAD, HC), jnp.float32),
            jax.ShapeDtypeStruct((2, DR, HC), jnp.float32),
        ],
        scratch_types=[
            pltpu.VMEM((CHUNK,), jnp.int32),
            pltpu.VMEM((CHUNK,), jnp.int32),
            pltpu.VMEM((CHUNK,), jnp.int32),
            pltpu.VMEM((CHUNK * H,), jnp.float32),
            pltpu.VMEM((CHUNK, HC), jnp.float32),
            pltpu.VMEM((CHUNK, HC), jnp.float32),
            pltpu.VMEM_SHARED((NPAD, HC), jnp.float32),
            pltpu.VMEM_SHARED((DR, HC), jnp.float32),
            pltpu.SemaphoreType.DMA,
            pltpu.SemaphoreType.DMA,
        ],
    )
    zeros = jnp.zeros((CHUNK, HC), jnp.float32)
    return kfn(h, src, dst, tflat, zeros)


_R2 = 1000


def _postA_body(acc0_ref, acc1_ref, dp0_ref, dp1_ref, h_ref, asrc_ref,
                adst_ref, k_ref, bias_ref, b_ref, tmp_ref, stats_ref,
                s_ref, q_ref):
    i = pl.program_id(0)
    kval = k_ref[0, 0]
    m = acc0_ref[...] + acc1_ref[...]
    d4 = dp0_ref[...] + dp1_ref[...]
    z = asrc_ref[...] + adst_ref[...]
    z = jnp.maximum(z, 0.2 * z)
    t4 = jnp.exp(z - kval)
    bmat = b_ref[...]
    d128 = jnp.dot(d4 + t4, bmat, preferred_element_type=jnp.float32)
    t128 = jnp.dot(t4, bmat, preferred_element_type=jnp.float32)
    o = (m + t128 * h_ref[...]) / (d128 + 1e-16) + bias_ref[...]
    tmp_ref[...] = o
    ps = jnp.sum(o, axis=0, keepdims=True)
    pq = jnp.sum(o * o, axis=0, keepdims=True)
    prev_s = jnp.where(i == 0, jnp.zeros_like(ps), s_ref[...])
    prev_q = jnp.where(i == 0, jnp.zeros_like(pq), q_ref[...])
    s_ref[...] = prev_s + ps
    q_ref[...] = prev_q + pq
    stats_ref[0:1, :] = s_ref[...]
    stats_ref[1:2, :] = q_ref[...]


def _tc_postA(acc0, acc1, dp0, dp1, h, asrc, adst, kmat, bias, Bmat):
    return pl.pallas_call(
        _postA_body,
        grid=(N // _R2,),
        in_specs=[
            pl.BlockSpec((_R2, HC), lambda i: (i, 0)),
            pl.BlockSpec((_R2, HC), lambda i: (i, 0)),
            pl.BlockSpec((_R2, H), lambda i: (i, 0)),
            pl.BlockSpec((_R2, H), lambda i: (i, 0)),
            pl.BlockSpec((_R2, HC), lambda i: (i, 0)),
            pl.BlockSpec((_R2, H), lambda i: (i, 0)),
            pl.BlockSpec((_R2, H), lambda i: (i, 0)),
            pl.BlockSpec((8, HC), lambda i: (0, 0)),
            pl.BlockSpec((HC,), lambda i: (0,)),
            pl.BlockSpec((H, HC), lambda i: (0, 0)),
        ],
        out_specs=[
            pl.BlockSpec((_R2, HC), lambda i: (i, 0)),
            pl.BlockSpec((2, HC), lambda i: (0, 0)),
        ],
        out_shape=[
            jax.ShapeDtypeStruct((N, HC), jnp.float32),
            jax.ShapeDtypeStruct((2, HC), jnp.float32),
        ],
        scratch_shapes=[
            pltpu.VMEM((1, HC), jnp.float32),
            pltpu.VMEM((1, HC), jnp.float32),
        ],
    )(acc0, acc1, dp0, dp1, h, asrc, adst, kmat, bias, Bmat)


def _postB_body(tmp_ref, stats_ref, gamma_ref, beta_ref, x_ref, y_ref):
    s = stats_ref[0:1, :]
    q = stats_ref[1:2, :]
    mu = s * (1.0 / N)
    var = q * (1.0 / N) - mu * mu
    inv = jax.lax.rsqrt(var + 1e-5)
    yv = (tmp_ref[...] - mu) * inv * gamma_ref[...] + beta_ref[...]
    y_ref[...] = jnp.maximum(yv, 0.0) + x_ref[...]


def _tc_postB(tmp, stats, gamma, beta, x):
    return pl.pallas_call(
        _postB_body,
        grid=(N // _R2,),
        in_specs=[
            pl.BlockSpec((_R2, HC), lambda i: (i, 0)),
            pl.BlockSpec((2, HC), lambda i: (0, 0)),
            pl.BlockSpec((HC,), lambda i: (0,)),
            pl.BlockSpec((HC,), lambda i: (0,)),
            pl.BlockSpec((_R2, HC), lambda i: (i, 0)),
        ],
        out_specs=pl.BlockSpec((_R2, HC), lambda i: (i, 0)),
        out_shape=jax.ShapeDtypeStruct((N, HC), jnp.float32),
    )(tmp, stats, gamma, beta, x)


def kernel(x, edge_index, W, att_src, att_dst, bias, gamma, beta):
    rows = jnp.arange(HC)
    heads = rows // C
    A_src = jnp.where(heads[:, None] == jnp.arange(H)[None, :],
                      att_src.reshape(-1)[:, None], 0.0).astype(jnp.float32)
    A_dst = jnp.where(heads[:, None] == jnp.arange(H)[None, :],
                      att_dst.reshape(-1)[:, None], 0.0).astype(jnp.float32)
    Bmat = (jnp.arange(H)[:, None] == heads[None, :]).astype(jnp.float32)

    h, asrc, adst, kmat = _tc_pre(x, W, A_src, A_dst)

    src = edge_index[0]
    dst = edge_index[1]
    k16 = kmat[0, :16]
    tflat = _sc_logits(asrc.reshape(-1), adst.reshape(-1), src, dst, k16)
    acc, accd = _sc_edges(h, src, dst, tflat)

    dp0 = accd[0].reshape(NPAD, H)[:N]
    dp1 = accd[1].reshape(NPAD, H)[:N]
    tmp, stats = _tc_postA(acc[0, :N], acc[1, :N], dp0, dp1, h, asrc, adst,
                           kmat, bias, Bmat)
    return _tc_postB(tmp, stats, gamma, beta, x)


# speedup vs baseline: 20.9479x; 1.1497x over previous
import dataclasses
import functools

import jax
import jax.numpy as jnp
from jax import lax
from jax.experimental import pallas as pl
from jax.experimental.pallas import tpu as pltpu
from jax.experimental.pallas import tpu_sc as plsc

N = 10000
E = 320000
F_IN = 128
H = 4
C = 32
HC = H * C
AW = HC + 16

NTILES = 32
EPW = E // NTILES
CHUNK = 80
NCHUNK = EPW // CHUNK
NPAD = 10240
RPT = NPAD // 16


_R1 = 1000


def _pre_body(x_ref, w_ref, as_ref, ad_ref, h_ref, asrc_ref, adst_ref,
              k_ref, mx_ref):
    i = pl.program_id(0)
    xb = x_ref[...]
    hb = jnp.dot(xb, w_ref[...], preferred_element_type=jnp.float32)
    h_ref[...] = hb
    a_s = jnp.dot(hb, as_ref[...], preferred_element_type=jnp.float32)
    a_d = jnp.dot(hb, ad_ref[...], preferred_element_type=jnp.float32)
    asrc_ref[...] = a_s
    adst_ref[...] = a_d
    prev_s = jnp.where(i == 0, -jnp.inf, mx_ref[0])
    prev_d = jnp.where(i == 0, -jnp.inf, mx_ref[1])
    ms = jnp.maximum(prev_s, jnp.max(a_s))
    md = jnp.maximum(prev_d, jnp.max(a_d))
    mx_ref[0] = ms
    mx_ref[1] = md
    m = ms + md
    k = jnp.maximum(m, 0.2 * m)
    k_ref[...] = jnp.full((8, HC), k, jnp.float32)


def _tc_pre(x, W, A_src, A_dst):
    return pl.pallas_call(
        _pre_body,
        grid=(N // _R1,),
        in_specs=[
            pl.BlockSpec((_R1, F_IN), lambda i: (i, 0)),
            pl.BlockSpec((F_IN, HC), lambda i: (0, 0)),
            pl.BlockSpec((HC, H), lambda i: (0, 0)),
            pl.BlockSpec((HC, H), lambda i: (0, 0)),
        ],
        out_specs=[
            pl.BlockSpec((_R1, HC), lambda i: (i, 0)),
            pl.BlockSpec((_R1, H), lambda i: (i, 0)),
            pl.BlockSpec((_R1, H), lambda i: (i, 0)),
            pl.BlockSpec((8, HC), lambda i: (0, 0)),
        ],
        out_shape=[
            jax.ShapeDtypeStruct((N, HC), jnp.float32),
            jax.ShapeDtypeStruct((N, H), jnp.float32),
            jax.ShapeDtypeStruct((N, H), jnp.float32),
            jax.ShapeDtypeStruct((8, HC), jnp.float32),
        ],
        scratch_shapes=[pltpu.SMEM((2,), jnp.float32)],
    )(x, W, A_src, A_dst)


DR = NPAD // 32

_SC_MESH = dict(core_axis_name="c", subcore_axis_name="s")


def _sc_cp():
    cp = pltpu.CompilerParams()
    if "needs_layout_passes" in pltpu.CompilerParams.__dataclass_fields__:
        cp = dataclasses.replace(cp, needs_layout_passes=False)
    return cp


def _sc_logits_body(asrc_hbm, adst_hbm, src_hbm, dst_hbm, k_hbm, t_hbm,
                    asrc_v, adst_v, kv, srci, dsti, tbuf, sem):
    cid = lax.axis_index("c")
    sid = lax.axis_index("s")
    iota16 = lax.iota(jnp.int32, 16)

    pltpu.sync_copy(asrc_hbm, asrc_v)
    pltpu.sync_copy(adst_hbm, adst_v)
    pltpu.sync_copy(k_hbm, kv)
    kvec = kv[...]
    base = (cid * 16 + sid) * EPW

    @pl.loop(0, NCHUNK)
    def _chunk(cn):
        off = base + cn * CHUNK
        pltpu.sync_copy(src_hbm.at[pl.ds(off, CHUNK)], srci)
        pltpu.sync_copy(dst_hbm.at[pl.ds(off, CHUNK)], dsti)

        @plsc.parallel_loop(0, CHUNK // 16, unroll=2)
        def _grp(g):
            s16 = srci[pl.ds(g * 16, 16)]
            d16 = dsti[pl.ds(g * 16, 16)]
            rows4 = (g * 16 + iota16) * H
            s4 = s16 * H
            d4 = d16 * H
            for hh in range(H):
                a_s = plsc.load_gather(asrc_v, [s4 + hh])
                a_d = plsc.load_gather(adst_v, [d4 + hh])
                e = a_s + a_d
                e = jnp.maximum(e, 0.2 * e)
                t = jnp.exp(e - kvec)
                plsc.store_scatter(tbuf, [rows4 + hh], t)

        pltpu.sync_copy(tbuf, t_hbm.at[pl.ds(off * H, CHUNK * H)])


def _sc_logits(asrc_flat, adst_flat, src, dst, k16):
    kfn = pl.kernel(
        _sc_logits_body,
        mesh=plsc.VectorSubcoreMesh(**_SC_MESH),
        compiler_params=_sc_cp(),
        out_type=jax.ShapeDtypeStruct((E * H,), jnp.float32),
        scratch_types=[
            pltpu.VMEM((N * H,), jnp.float32),
            pltpu.VMEM((N * H,), jnp.float32),
            pltpu.VMEM((16,), jnp.float32),
            pltpu.VMEM((CHUNK,), jnp.int32),
            pltpu.VMEM((CHUNK,), jnp.int32),
            pltpu.VMEM((CHUNK * H,), jnp.float32),
            pltpu.SemaphoreType.DMA,
        ],
    )
    return kfn(asrc_flat, adst_flat, src, dst, k16)


def _sc_body(h_hbm, src_hbm, dst_hbm, t_hbm, z_hbm,
             out_hbm, outd_hbm, srci, dsti, dsti2, tbuf,
             gin, uout, tden, acc, accd, sem_g, sem_s):
    cid = lax.axis_index("c")
    sid = lax.axis_index("s")
    iota16 = lax.iota(jnp.int32, 16)

    pltpu.sync_copy(z_hbm, tden)

    @pl.loop(0, RPT // CHUNK)
    def _za(j):
        pltpu.sync_copy(z_hbm, acc.at[pl.ds(sid * RPT + j * CHUNK, CHUNK)])

    @pl.when(sid < DR // 32)
    def _zd():
        pltpu.sync_copy(z_hbm.at[pl.ds(0, 32)], accd.at[pl.ds(sid * 32, 32)])

    plsc.subcore_barrier()

    zero16 = jnp.zeros((16,), jnp.float32)
    base = (cid * 16 + sid) * EPW

    @pl.loop(0, NCHUNK)
    def _chunk(cn):
        off = base + cn * CHUNK
        pltpu.sync_copy(src_hbm.at[pl.ds(off, CHUNK)], srci)
        pltpu.sync_copy(dst_hbm.at[pl.ds(off, CHUNK)], dsti)
        pltpu.sync_copy(t_hbm.at[pl.ds(off * H, CHUNK * H)], tbuf)
        pltpu.async_copy(h_hbm.at[srci], gin, sem_g).wait()

        @plsc.parallel_loop(0, CHUNK // 16, unroll=2)
        def _grp(g):
            d16 = dsti[pl.ds(g * 16, 16)]
            rows = g * 16 + iota16
            rows4 = rows * H
            cdt = (d16 & 31) * H
            dsti2[pl.ds(g * 16, 16)] = lax.shift_right_logical(d16, 5)
            for hh in range(H):
                t = plsc.load_gather(tbuf, [rows4 + hh])
                plsc.store_scatter(tden, [rows, cdt + hh], t)
                for cc in range(hh * C, (hh + 1) * C):
                    colv = jnp.full((16,), cc, jnp.int32)
                    v = plsc.load_gather(gin, [rows, colv])
                    plsc.store_scatter(uout, [rows, colv], v * t)

        pltpu.sync_copy(uout, acc.at[dsti], add=True)
        pltpu.sync_copy(tden, accd.at[dsti2], add=True)


        @pl.loop(0, CHUNK // 16)
        def _rz(g):
            d16 = dsti[pl.ds(g * 16, 16)]
            rows = g * 16 + iota16
            cdt = (d16 & 31) * H
            for hh in range(H):
                plsc.store_scatter(tden, [rows, cdt + hh], zero16)

    plsc.subcore_barrier()
    pltpu.sync_copy(acc.at[pl.ds(sid * RPT, RPT)],
                    out_hbm.at[cid, pl.ds(sid * RPT, RPT)])

    @pl.when(sid == 0)
    def _dd():
        pltpu.sync_copy(accd, outd_hbm.at[cid])


def _sc_edges(h, src, dst, tflat):
    kfn = pl.kernel(
        _sc_body,
        mesh=plsc.VectorSubcoreMesh(**_SC_MESH),
        compiler_params=_sc_cp(),
        out_type=[
            jax.ShapeDtypeStruct((2, NPAD, HC), jnp.float32),
            jax.ShapeDtypeStruct((2, DR, HC), jnp.float32),
        ],
        scratch_types=[
            pltpu.VMEM((CHUNK,), jnp.int32),
            pltpu.VMEM((CHUNK,), jnp.int32),
            pltpu.VMEM((CHUNK,), jnp.int32),
            pltpu.VMEM((CHUNK * H,), jnp.float32),
            pltpu.VMEM((CHUNK, HC), jnp.float32),
            pltpu.VMEM((CHUNK, HC), jnp.float32),
            pltpu.VMEM((CHUNK, HC), jnp.float32),
            pltpu.VMEM_SHARED((NPAD, HC), jnp.float32),
            pltpu.VMEM_SHARED((DR, HC), jnp.float32),
            pltpu.SemaphoreType.DMA,
            pltpu.SemaphoreType.DMA,
        ],
    )
    zeros = jnp.zeros((CHUNK, HC), jnp.float32)
    return kfn(h, src, dst, tflat, zeros)


_R2 = 1000


def _postA_body(acc0_ref, acc1_ref, dp0_ref, dp1_ref, h_ref, asrc_ref,
                adst_ref, k_ref, bias_ref, b_ref, tmp_ref, stats_ref,
                s_ref, q_ref):
    i = pl.program_id(0)
    kval = k_ref[0, 0]
    m = acc0_ref[...] + acc1_ref[...]
    d4 = dp0_ref[...] + dp1_ref[...]
    z = asrc_ref[...] + adst_ref[...]
    z = jnp.maximum(z, 0.2 * z)
    t4 = jnp.exp(z - kval)
    bmat = b_ref[...]
    d128 = jnp.dot(d4 + t4, bmat, preferred_element_type=jnp.float32)
    t128 = jnp.dot(t4, bmat, preferred_element_type=jnp.float32)
    o = (m + t128 * h_ref[...]) / (d128 + 1e-16) + bias_ref[...]
    tmp_ref[...] = o
    ps = jnp.sum(o, axis=0, keepdims=True)
    pq = jnp.sum(o * o, axis=0, keepdims=True)
    prev_s = jnp.where(i == 0, jnp.zeros_like(ps), s_ref[...])
    prev_q = jnp.where(i == 0, jnp.zeros_like(pq), q_ref[...])
    s_ref[...] = prev_s + ps
    q_ref[...] = prev_q + pq
    stats_ref[0:1, :] = s_ref[...]
    stats_ref[1:2, :] = q_ref[...]


def _tc_postA(acc0, acc1, dp0, dp1, h, asrc, adst, kmat, bias, Bmat):
    return pl.pallas_call(
        _postA_body,
        grid=(N // _R2,),
        in_specs=[
            pl.BlockSpec((_R2, HC), lambda i: (i, 0)),
            pl.BlockSpec((_R2, HC), lambda i: (i, 0)),
            pl.BlockSpec((_R2, H), lambda i: (i, 0)),
            pl.BlockSpec((_R2, H), lambda i: (i, 0)),
            pl.BlockSpec((_R2, HC), lambda i: (i, 0)),
            pl.BlockSpec((_R2, H), lambda i: (i, 0)),
            pl.BlockSpec((_R2, H), lambda i: (i, 0)),
            pl.BlockSpec((8, HC), lambda i: (0, 0)),
            pl.BlockSpec((HC,), lambda i: (0,)),
            pl.BlockSpec((H, HC), lambda i: (0, 0)),
        ],
        out_specs=[
            pl.BlockSpec((_R2, HC), lambda i: (i, 0)),
            pl.BlockSpec((2, HC), lambda i: (0, 0)),
        ],
        out_shape=[
            jax.ShapeDtypeStruct((N, HC), jnp.float32),
            jax.ShapeDtypeStruct((2, HC), jnp.float32),
        ],
        scratch_shapes=[
            pltpu.VMEM((1, HC), jnp.float32),
            pltpu.VMEM((1, HC), jnp.float32),
        ],
    )(acc0, acc1, dp0, dp1, h, asrc, adst, kmat, bias, Bmat)


def _postB_body(tmp_ref, stats_ref, gamma_ref, beta_ref, x_ref, y_ref):
    s = stats_ref[0:1, :]
    q = stats_ref[1:2, :]
    mu = s * (1.0 / N)
    var = q * (1.0 / N) - mu * mu
    inv = jax.lax.rsqrt(var + 1e-5)
    yv = (tmp_ref[...] - mu) * inv * gamma_ref[...] + beta_ref[...]
    y_ref[...] = jnp.maximum(yv, 0.0) + x_ref[...]


def _tc_postB(tmp, stats, gamma, beta, x):
    return pl.pallas_call(
        _postB_body,
        grid=(N // _R2,),
        in_specs=[
            pl.BlockSpec((_R2, HC), lambda i: (i, 0)),
            pl.BlockSpec((2, HC), lambda i: (0, 0)),
            pl.BlockSpec((HC,), lambda i: (0,)),
            pl.BlockSpec((HC,), lambda i: (0,)),
            pl.BlockSpec((_R2, HC), lambda i: (i, 0)),
        ],
        out_specs=pl.BlockSpec((_R2, HC), lambda i: (i, 0)),
        out_shape=jax.ShapeDtypeStruct((N, HC), jnp.float32),
    )(tmp, stats, gamma, beta, x)


def kernel(x, edge_index, W, att_src, att_dst, bias, gamma, beta):
    rows = jnp.arange(HC)
    heads = rows // C
    A_src = jnp.where(heads[:, None] == jnp.arange(H)[None, :],
                      att_src.reshape(-1)[:, None], 0.0).astype(jnp.float32)
    A_dst = jnp.where(heads[:, None] == jnp.arange(H)[None, :],
                      att_dst.reshape(-1)[:, None], 0.0).astype(jnp.float32)
    Bmat = (jnp.arange(H)[:, None] == heads[None, :]).astype(jnp.float32)

    h, asrc, adst, kmat = _tc_pre(x, W, A_src, A_dst)

    src = edge_index[0]
    dst = edge_index[1]
    k16 = kmat[0, :16]
    tflat = _sc_logits(asrc.reshape(-1), adst.reshape(-1), src, dst, k16)
    acc, accd = _sc_edges(h, src, dst, tflat)

    dp0 = accd[0].reshape(NPAD, H)[:N]
    dp1 = accd[1].reshape(NPAD, H)[:N]
    tmp, stats = _tc_postA(acc[0, :N], acc[1, :N], dp0, dp1, h, asrc, adst,
                           kmat, bias, Bmat)
    return _tc_postB(tmp, stats, gamma, beta, x)


# speedup vs baseline: 24.4505x; 1.1672x over previous
import dataclasses
import functools

import jax
import jax.numpy as jnp
from jax import lax
from jax.experimental import pallas as pl
from jax.experimental.pallas import tpu as pltpu
from jax.experimental.pallas import tpu_sc as plsc

N = 10000
E = 320000
F_IN = 128
H = 4
C = 32
HC = H * C
AW = HC + 16

NTILES = 32
EPW = E // NTILES
CHUNK = 80
NCHUNK = EPW // CHUNK
NPAD = 10240
RPT = NPAD // 16


_R1 = 1000


def _pre_body(x_ref, w_ref, as_ref, ad_ref, h_ref, asrc_ref, adst_ref,
              k_ref, mx_ref):
    i = pl.program_id(0)
    xb = x_ref[...]
    hb = jnp.dot(xb, w_ref[...], preferred_element_type=jnp.float32)
    h_ref[...] = hb
    a_s = jnp.dot(hb, as_ref[...], preferred_element_type=jnp.float32)
    a_d = jnp.dot(hb, ad_ref[...], preferred_element_type=jnp.float32)
    asrc_ref[...] = a_s
    adst_ref[...] = a_d
    prev_s = jnp.where(i == 0, -jnp.inf, mx_ref[0])
    prev_d = jnp.where(i == 0, -jnp.inf, mx_ref[1])
    ms = jnp.maximum(prev_s, jnp.max(a_s))
    md = jnp.maximum(prev_d, jnp.max(a_d))
    mx_ref[0] = ms
    mx_ref[1] = md
    m = ms + md
    k = jnp.maximum(m, 0.2 * m)
    k_ref[...] = jnp.full((8, HC), k, jnp.float32)


def _tc_pre(x, W, A_src, A_dst):
    return pl.pallas_call(
        _pre_body,
        grid=(N // _R1,),
        in_specs=[
            pl.BlockSpec((_R1, F_IN), lambda i: (i, 0)),
            pl.BlockSpec((F_IN, HC), lambda i: (0, 0)),
            pl.BlockSpec((HC, H), lambda i: (0, 0)),
            pl.BlockSpec((HC, H), lambda i: (0, 0)),
        ],
        out_specs=[
            pl.BlockSpec((_R1, HC), lambda i: (i, 0)),
            pl.BlockSpec((_R1, H), lambda i: (i, 0)),
            pl.BlockSpec((_R1, H), lambda i: (i, 0)),
            pl.BlockSpec((8, HC), lambda i: (0, 0)),
        ],
        out_shape=[
            jax.ShapeDtypeStruct((N, HC), jnp.float32),
            jax.ShapeDtypeStruct((N, H), jnp.float32),
            jax.ShapeDtypeStruct((N, H), jnp.float32),
            jax.ShapeDtypeStruct((8, HC), jnp.float32),
        ],
        scratch_shapes=[pltpu.SMEM((2,), jnp.float32)],
    )(x, W, A_src, A_dst)


DR = NPAD // 32

_SC_MESH = dict(core_axis_name="c", subcore_axis_name="s")


def _sc_cp():
    cp = pltpu.CompilerParams()
    if "needs_layout_passes" in pltpu.CompilerParams.__dataclass_fields__:
        cp = dataclasses.replace(cp, needs_layout_passes=False)
    return cp


def _sc_logits_body(asrc_hbm, adst_hbm, src_hbm, dst_hbm, k_hbm, t_hbm,
                    asrc_v, adst_v, kv, srci, dsti, tbuf, sem):
    cid = lax.axis_index("c")
    sid = lax.axis_index("s")
    iota16 = lax.iota(jnp.int32, 16)

    pltpu.sync_copy(asrc_hbm, asrc_v)
    pltpu.sync_copy(adst_hbm, adst_v)
    pltpu.sync_copy(k_hbm, kv)
    kvec = kv[...]
    base = (cid * 16 + sid) * EPW

    @pl.loop(0, NCHUNK)
    def _chunk(cn):
        off = base + cn * CHUNK
        pltpu.sync_copy(src_hbm.at[pl.ds(off, CHUNK)], srci)
        pltpu.sync_copy(dst_hbm.at[pl.ds(off, CHUNK)], dsti)

        @plsc.parallel_loop(0, CHUNK // 16, unroll=2)
        def _grp(g):
            s16 = srci[pl.ds(g * 16, 16)]
            d16 = dsti[pl.ds(g * 16, 16)]
            rows4 = (g * 16 + iota16) * H
            s4 = s16 * H
            d4 = d16 * H
            for hh in range(H):
                a_s = plsc.load_gather(asrc_v, [s4 + hh])
                a_d = plsc.load_gather(adst_v, [d4 + hh])
                e = a_s + a_d
                e = jnp.maximum(e, 0.2 * e)
                t = jnp.exp(e - kvec)
                plsc.store_scatter(tbuf, [rows4 + hh], t)

        pltpu.sync_copy(tbuf, t_hbm.at[pl.ds(off * H, CHUNK * H)])


def _sc_logits(asrc_flat, adst_flat, src, dst, k16):
    kfn = pl.kernel(
        _sc_logits_body,
        mesh=plsc.VectorSubcoreMesh(**_SC_MESH),
        compiler_params=_sc_cp(),
        out_type=jax.ShapeDtypeStruct((E * H,), jnp.float32),
        scratch_types=[
            pltpu.VMEM((N * H,), jnp.float32),
            pltpu.VMEM((N * H,), jnp.float32),
            pltpu.VMEM((16,), jnp.float32),
            pltpu.VMEM((CHUNK,), jnp.int32),
            pltpu.VMEM((CHUNK,), jnp.int32),
            pltpu.VMEM((CHUNK * H,), jnp.float32),
            pltpu.SemaphoreType.DMA,
        ],
    )
    return kfn(asrc_flat, adst_flat, src, dst, k16)


def _sc_body(h_hbm, src_hbm, dst_hbm, t_hbm, z_hbm,
             out_hbm, outd_hbm, srci, dsti, dsti2, tbuf,
             gin, uout, tden, acc, accd, sem_g, sem_s):
    cid = lax.axis_index("c")
    sid = lax.axis_index("s")
    iota16 = lax.iota(jnp.int32, 16)

    pltpu.sync_copy(z_hbm, tden)

    @pl.loop(0, RPT // CHUNK)
    def _za(j):
        pltpu.sync_copy(z_hbm, acc.at[pl.ds(sid * RPT + j * CHUNK, CHUNK)])

    @pl.when(sid < DR // 32)
    def _zd():
        pltpu.sync_copy(z_hbm.at[pl.ds(0, 32)], accd.at[pl.ds(sid * 32, 32)])

    plsc.subcore_barrier()

    zero16 = jnp.zeros((16,), jnp.float32)
    base = (cid * 16 + sid) * EPW

    @pl.loop(0, NCHUNK)
    def _chunk(cn):
        off = base + cn * CHUNK
        pltpu.sync_copy(src_hbm.at[pl.ds(off, CHUNK)], srci)
        pltpu.sync_copy(dst_hbm.at[pl.ds(off, CHUNK)], dsti)
        pltpu.sync_copy(t_hbm.at[pl.ds(off * H, CHUNK * H)], tbuf)
        pltpu.async_copy(h_hbm.at[srci], gin, sem_g).wait()

        @plsc.parallel_loop(0, CHUNK // 16, unroll=2)
        def _grp(g):
            d16 = dsti[pl.ds(g * 16, 16)]
            rows = g * 16 + iota16
            rows4 = rows * H
            cdt = (d16 & 31) * H
            dsti2[pl.ds(g * 16, 16)] = lax.shift_right_logical(d16, 5)
            for hh in range(H):
                t = plsc.load_gather(tbuf, [rows4 + hh])
                plsc.store_scatter(tden, [rows, cdt + hh], t)
                scaled = []
                for cc in range(hh * C, (hh + 1) * C):
                    colv = jnp.full((16,), cc, jnp.int32)
                    v = plsc.load_gather(gin, [rows, colv])
                    scaled.append(v * t)
                for k, cc in enumerate(range(hh * C, (hh + 1) * C)):
                    colv = jnp.full((16,), cc, jnp.int32)
                    plsc.store_scatter(uout, [rows, colv], scaled[k])

        pltpu.sync_copy(uout, acc.at[dsti], add=True)
        pltpu.sync_copy(tden, accd.at[dsti2], add=True)


        @pl.loop(0, CHUNK // 16)
        def _rz(g):
            d16 = dsti[pl.ds(g * 16, 16)]
            rows = g * 16 + iota16
            cdt = (d16 & 31) * H
            for hh in range(H):
                plsc.store_scatter(tden, [rows, cdt + hh], zero16)

    plsc.subcore_barrier()
    pltpu.sync_copy(acc.at[pl.ds(sid * RPT, RPT)],
                    out_hbm.at[cid, pl.ds(sid * RPT, RPT)])

    @pl.when(sid == 0)
    def _dd():
        pltpu.sync_copy(accd, outd_hbm.at[cid])


def _sc_edges(h, src, dst, tflat):
    kfn = pl.kernel(
        _sc_body,
        mesh=plsc.VectorSubcoreMesh(**_SC_MESH),
        compiler_params=_sc_cp(),
        out_type=[
            jax.ShapeDtypeStruct((2, NPAD, HC), jnp.float32),
            jax.ShapeDtypeStruct((2, DR, HC), jnp.float32),
        ],
        scratch_types=[
            pltpu.VMEM((CHUNK,), jnp.int32),
            pltpu.VMEM((CHUNK,), jnp.int32),
            pltpu.VMEM((CHUNK,), jnp.int32),
            pltpu.VMEM((CHUNK * H,), jnp.float32),
            pltpu.VMEM((CHUNK, HC), jnp.float32),
            pltpu.VMEM((CHUNK, HC), jnp.float32),
            pltpu.VMEM((CHUNK, HC), jnp.float32),
            pltpu.VMEM_SHARED((NPAD, HC), jnp.float32),
            pltpu.VMEM_SHARED((DR, HC), jnp.float32),
            pltpu.SemaphoreType.DMA,
            pltpu.SemaphoreType.DMA,
        ],
    )
    zeros = jnp.zeros((CHUNK, HC), jnp.float32)
    return kfn(h, src, dst, tflat, zeros)


_R2 = 1000


def _postA_body(acc0_ref, acc1_ref, dp0_ref, dp1_ref, h_ref, asrc_ref,
                adst_ref, k_ref, bias_ref, b_ref, tmp_ref, stats_ref,
                s_ref, q_ref):
    i = pl.program_id(0)
    kval = k_ref[0, 0]
    m = acc0_ref[...] + acc1_ref[...]
    d4 = dp0_ref[...] + dp1_ref[...]
    z = asrc_ref[...] + adst_ref[...]
    z = jnp.maximum(z, 0.2 * z)
    t4 = jnp.exp(z - kval)
    bmat = b_ref[...]
    d128 = jnp.dot(d4 + t4, bmat, preferred_element_type=jnp.float32)
    t128 = jnp.dot(t4, bmat, preferred_element_type=jnp.float32)
    o = (m + t128 * h_ref[...]) / (d128 + 1e-16) + bias_ref[...]
    tmp_ref[...] = o
    ps = jnp.sum(o, axis=0, keepdims=True)
    pq = jnp.sum(o * o, axis=0, keepdims=True)
    prev_s = jnp.where(i == 0, jnp.zeros_like(ps), s_ref[...])
    prev_q = jnp.where(i == 0, jnp.zeros_like(pq), q_ref[...])
    s_ref[...] = prev_s + ps
    q_ref[...] = prev_q + pq
    stats_ref[0:1, :] = s_ref[...]
    stats_ref[1:2, :] = q_ref[...]


def _tc_postA(acc0, acc1, dp0, dp1, h, asrc, adst, kmat, bias, Bmat):
    return pl.pallas_call(
        _postA_body,
        grid=(N // _R2,),
        in_specs=[
            pl.BlockSpec((_R2, HC), lambda i: (i, 0)),
            pl.BlockSpec((_R2, HC), lambda i: (i, 0)),
            pl.BlockSpec((_R2, H), lambda i: (i, 0)),
            pl.BlockSpec((_R2, H), lambda i: (i, 0)),
            pl.BlockSpec((_R2, HC), lambda i: (i, 0)),
            pl.BlockSpec((_R2, H), lambda i: (i, 0)),
            pl.BlockSpec((_R2, H), lambda i: (i, 0)),
            pl.BlockSpec((8, HC), lambda i: (0, 0)),
            pl.BlockSpec((HC,), lambda i: (0,)),
            pl.BlockSpec((H, HC), lambda i: (0, 0)),
        ],
        out_specs=[
            pl.BlockSpec((_R2, HC), lambda i: (i, 0)),
            pl.BlockSpec((2, HC), lambda i: (0, 0)),
        ],
        out_shape=[
            jax.ShapeDtypeStruct((N, HC), jnp.float32),
            jax.ShapeDtypeStruct((2, HC), jnp.float32),
        ],
        scratch_shapes=[
            pltpu.VMEM((1, HC), jnp.float32),
            pltpu.VMEM((1, HC), jnp.float32),
        ],
    )(acc0, acc1, dp0, dp1, h, asrc, adst, kmat, bias, Bmat)


def _postB_body(tmp_ref, stats_ref, gamma_ref, beta_ref, x_ref, y_ref):
    s = stats_ref[0:1, :]
    q = stats_ref[1:2, :]
    mu = s * (1.0 / N)
    var = q * (1.0 / N) - mu * mu
    inv = jax.lax.rsqrt(var + 1e-5)
    yv = (tmp_ref[...] - mu) * inv * gamma_ref[...] + beta_ref[...]
    y_ref[...] = jnp.maximum(yv, 0.0) + x_ref[...]


def _tc_postB(tmp, stats, gamma, beta, x):
    return pl.pallas_call(
        _postB_body,
        grid=(N // _R2,),
        in_specs=[
            pl.BlockSpec((_R2, HC), lambda i: (i, 0)),
            pl.BlockSpec((2, HC), lambda i: (0, 0)),
            pl.BlockSpec((HC,), lambda i: (0,)),
            pl.BlockSpec((HC,), lambda i: (0,)),
            pl.BlockSpec((_R2, HC), lambda i: (i, 0)),
        ],
        out_specs=pl.BlockSpec((_R2, HC), lambda i: (i, 0)),
        out_shape=jax.ShapeDtypeStruct((N, HC), jnp.float32),
    )(tmp, stats, gamma, beta, x)


def kernel(x, edge_index, W, att_src, att_dst, bias, gamma, beta):
    rows = jnp.arange(HC)
    heads = rows // C
    A_src = jnp.where(heads[:, None] == jnp.arange(H)[None, :],
                      att_src.reshape(-1)[:, None], 0.0).astype(jnp.float32)
    A_dst = jnp.where(heads[:, None] == jnp.arange(H)[None, :],
                      att_dst.reshape(-1)[:, None], 0.0).astype(jnp.float32)
    Bmat = (jnp.arange(H)[:, None] == heads[None, :]).astype(jnp.float32)

    h, asrc, adst, kmat = _tc_pre(x, W, A_src, A_dst)

    src = edge_index[0]
    dst = edge_index[1]
    k16 = kmat[0, :16]
    tflat = _sc_logits(asrc.reshape(-1), adst.reshape(-1), src, dst, k16)
    acc, accd = _sc_edges(h, src, dst, tflat)

    dp0 = accd[0].reshape(NPAD, H)[:N]
    dp1 = accd[1].reshape(NPAD, H)[:N]
    tmp, stats = _tc_postA(acc[0, :N], acc[1, :N], dp0, dp1, h, asrc, adst,
                           kmat, bias, Bmat)
    return _tc_postB(tmp, stats, gamma, beta, x)


# speedup vs baseline: 63.1892x; 2.5844x over previous
import dataclasses
import functools

import jax
import jax.numpy as jnp
from jax import lax
from jax.experimental import pallas as pl
from jax.experimental.pallas import tpu as pltpu
from jax.experimental.pallas import tpu_sc as plsc

N = 10000
E = 320000
F_IN = 128
H = 4
C = 32
HC = H * C

NTILES = 32
EPW = 10240
E_PAD = EPW * NTILES
PADDST = 10100
TP = 1024
CHUNK = 80
NCHUNK = EPW // CHUNK
CHUNK1 = 512
NCHUNK1 = EPW // CHUNK1
NPAD = 10240
RPT = NPAD // 16


_R1 = 1000


def _pre_body(x_ref, w_ref, as_ref, ad_ref, h_ref, asrc_ref, adst_ref,
              k_ref, mx_ref):
    i = pl.program_id(0)
    xb = x_ref[...]
    hb = jnp.dot(xb, w_ref[...], preferred_element_type=jnp.float32)
    h_ref[...] = hb
    a_s = jnp.dot(hb, as_ref[...], preferred_element_type=jnp.float32)
    a_d = jnp.dot(hb, ad_ref[...], preferred_element_type=jnp.float32)
    asrc_ref[...] = a_s
    adst_ref[...] = a_d
    prev_s = jnp.where(i == 0, -jnp.inf, mx_ref[0])
    prev_d = jnp.where(i == 0, -jnp.inf, mx_ref[1])
    ms = jnp.maximum(prev_s, jnp.max(a_s))
    md = jnp.maximum(prev_d, jnp.max(a_d))
    mx_ref[0] = ms
    mx_ref[1] = md
    m = ms + md
    k = jnp.maximum(m, 0.2 * m)
    k_ref[...] = jnp.full((8, HC), k, jnp.float32)


def _tc_pre(x, W, A_src, A_dst):
    return pl.pallas_call(
        _pre_body,
        grid=(N // _R1,),
        in_specs=[
            pl.BlockSpec((_R1, F_IN), lambda i: (i, 0)),
            pl.BlockSpec((F_IN, HC), lambda i: (0, 0)),
            pl.BlockSpec((HC, H), lambda i: (0, 0)),
            pl.BlockSpec((HC, H), lambda i: (0, 0)),
        ],
        out_specs=[
            pl.BlockSpec((_R1, HC), lambda i: (i, 0)),
            pl.BlockSpec((_R1, H), lambda i: (i, 0)),
            pl.BlockSpec((_R1, H), lambda i: (i, 0)),
            pl.BlockSpec((8, HC), lambda i: (0, 0)),
        ],
        out_shape=[
            jax.ShapeDtypeStruct((N, HC), jnp.float32),
            jax.ShapeDtypeStruct((N, H), jnp.float32),
            jax.ShapeDtypeStruct((N, H), jnp.float32),
            jax.ShapeDtypeStruct((8, HC), jnp.float32),
        ],
        scratch_shapes=[pltpu.SMEM((2,), jnp.float32)],
    )(x, W, A_src, A_dst)


DR = NPAD // 32

_SC_MESH = dict(core_axis_name="c", subcore_axis_name="s")


def _sc_cp():
    cp = pltpu.CompilerParams()
    if "needs_layout_passes" in pltpu.CompilerParams.__dataclass_fields__:
        cp = dataclasses.replace(cp, needs_layout_passes=False)
    return cp


def _sc_logits_body(asrc_hbm, adst_hbm, src_hbm, dst_hbm, k_hbm, t_hbm,
                    asrc_v, adst_v, kv, srci, dsti, tbuf, sem):
    cid = lax.axis_index("c")
    sid = lax.axis_index("s")
    iota16 = lax.iota(jnp.int32, 16)

    pltpu.sync_copy(asrc_hbm, asrc_v)
    pltpu.sync_copy(adst_hbm, adst_v)
    pltpu.sync_copy(k_hbm, kv)
    kvec = kv[...]
    base = (cid * 16 + sid) * EPW

    @pl.loop(0, NCHUNK1)
    def _chunk(cn):
        off = base + cn * CHUNK1
        pltpu.sync_copy(src_hbm.at[pl.ds(off, CHUNK1)], srci)
        pltpu.sync_copy(dst_hbm.at[pl.ds(off, CHUNK1)], dsti)

        @plsc.parallel_loop(0, CHUNK1 // 16, unroll=2)
        def _grp(g):
            s16 = srci[pl.ds(g * 16, 16)]
            d16 = dsti[pl.ds(g * 16, 16)]
            rows4 = (g * 16 + iota16) * H
            s4 = s16 * H
            d4 = d16 * H
            for hh in range(H):
                a_s = plsc.load_gather(asrc_v, [s4 + hh])
                a_d = plsc.load_gather(adst_v, [d4 + hh])
                e = a_s + a_d
                e = jnp.maximum(e, 0.2 * e)
                t = jnp.exp(e - kvec)
                plsc.store_scatter(tbuf, [rows4 + hh], t)

        pltpu.sync_copy(tbuf, t_hbm.at[pl.ds(off * H, CHUNK1 * H)])


def _sc_logits(asrc_flat, adst_flat, src, dst, k16):
    kfn = pl.kernel(
        _sc_logits_body,
        mesh=plsc.VectorSubcoreMesh(**_SC_MESH),
        compiler_params=_sc_cp(),
        out_type=jax.ShapeDtypeStruct((E_PAD * H,), jnp.float32),
        scratch_types=[
            pltpu.VMEM((N * H + TP,), jnp.float32),
            pltpu.VMEM((N * H + TP,), jnp.float32),
            pltpu.VMEM((16,), jnp.float32),
            pltpu.VMEM((CHUNK1,), jnp.int32),
            pltpu.VMEM((CHUNK1,), jnp.int32),
            pltpu.VMEM((CHUNK1 * H,), jnp.float32),
            pltpu.SemaphoreType.DMA,
        ],
    )
    return kfn(asrc_flat, adst_flat, src, dst, k16)


def _sc_body(h_hbm, src_hbm, dst_hbm, t_hbm, z_hbm,
             out_hbm, outd_hbm, srci0, srci1, dsti0, dsti1, tbuf0, tbuf1,
             gin0, gin1, dsti2, uout, tden, acc, accd, sem_g, sem_i,
             sem_s):
    cid = lax.axis_index("c")
    sid = lax.axis_index("s")
    iota16 = lax.iota(jnp.int32, 16)

    pltpu.sync_copy(z_hbm, tden)

    @pl.loop(0, RPT // CHUNK)
    def _za(j):
        pltpu.sync_copy(z_hbm, acc.at[pl.ds(sid * RPT + j * CHUNK, CHUNK)])

    @pl.when(sid < DR // 32)
    def _zd():
        pltpu.sync_copy(z_hbm.at[pl.ds(0, 32)], accd.at[pl.ds(sid * 32, 32)])

    plsc.subcore_barrier()

    zero16 = jnp.zeros((16,), jnp.float32)
    base = (cid * 16 + sid) * EPW
    bufs = ((srci0, dsti0, tbuf0, gin0), (srci1, dsti1, tbuf1, gin1))

    def idx_start(c, bid):
        srci, dsti, tbuf, _ = bufs[bid]
        off = base + c * CHUNK
        cp1 = pltpu.async_copy(src_hbm.at[pl.ds(off, CHUNK)], srci, sem_i)
        cp2 = pltpu.async_copy(dst_hbm.at[pl.ds(off, CHUNK)], dsti, sem_i)
        cp3 = pltpu.async_copy(t_hbm.at[pl.ds(off * H, CHUNK * H)], tbuf,
                               sem_i)
        return cp1, cp2, cp3

    def idx_wait(c, bid):
        srci, dsti, tbuf, _ = bufs[bid]
        off = base + c * CHUNK
        pltpu.make_async_copy(src_hbm.at[pl.ds(off, CHUNK)], srci,
                              sem_i).wait()
        pltpu.make_async_copy(dst_hbm.at[pl.ds(off, CHUNK)], dsti,
                              sem_i).wait()
        pltpu.make_async_copy(t_hbm.at[pl.ds(off * H, CHUNK * H)], tbuf,
                              sem_i).wait()

    def gather_start(bid):
        srci, _, _, gin = bufs[bid]
        pltpu.async_copy(h_hbm.at[srci], gin, sem_g)

    def gather_wait(bid):
        srci, _, _, gin = bufs[bid]
        pltpu.make_async_copy(h_hbm.at[srci], gin, sem_g).wait()

    idx_start(0, 0)
    idx_wait(0, 0)
    gather_start(0)
    idx_start(1, 1)

    def pipe_iter(c, p):
        srci, dsti, tbuf, gin = bufs[p]
        q = 1 - p

        @pl.when(c + 1 < NCHUNK)
        def _nx():
            idx_wait(c + 1, q)
            gather_start(q)

        gather_wait(p)

        @plsc.parallel_loop(0, CHUNK // H, unroll=2)
        def _equad(j):
            tq = tbuf[pl.ds(j * 16, 16)]
            for k in range(4):
                ee = j * H + k
                for hh in range(H):
                    tb = jnp.full((16,), tq[4 * k + hh], jnp.float32)
                    for half in range(2):
                        cc = hh * C + half * 16
                        v = gin[ee, pl.ds(cc, 16)]
                        uout[ee, pl.ds(cc, 16)] = v * tb

        @plsc.parallel_loop(0, CHUNK // 16, unroll=2)
        def _grp(g):
            d16 = dsti[pl.ds(g * 16, 16)]
            rows = g * 16 + iota16
            rows4 = rows * H
            cdt = (d16 & 31) * H
            dsti2[pl.ds(g * 16, 16)] = lax.shift_right_logical(d16, 5)
            for hh in range(H):
                t = plsc.load_gather(tbuf, [rows4 + hh])
                plsc.store_scatter(tden, [rows, cdt + hh], t)

        pltpu.sync_copy(uout, acc.at[dsti], add=True)
        pltpu.sync_copy(tden, accd.at[dsti2], add=True)

        @plsc.parallel_loop(0, CHUNK // 16, unroll=2)
        def _rz(g):
            d16 = dsti[pl.ds(g * 16, 16)]
            rows = g * 16 + iota16
            cdt = (d16 & 31) * H
            for hh in range(H):
                plsc.store_scatter(tden, [rows, cdt + hh], zero16)

        @pl.when(c + 2 < NCHUNK)
        def _pf():
            idx_start(c + 2, p)

    @pl.loop(0, NCHUNK, step=2)
    def _chunk(cn):
        pipe_iter(cn, 0)
        pipe_iter(cn + 1, 1)

    plsc.subcore_barrier()
    pltpu.sync_copy(acc.at[pl.ds(sid * RPT, RPT)],
                    out_hbm.at[cid, pl.ds(sid * RPT, RPT)])

    @pl.when(sid == 0)
    def _dd():
        pltpu.sync_copy(accd, outd_hbm.at[cid])


def _sc_edges(h, src, dst, tflat):
    kfn = pl.kernel(
        _sc_body,
        mesh=plsc.VectorSubcoreMesh(**_SC_MESH),
        compiler_params=_sc_cp(),
        out_type=[
            jax.ShapeDtypeStruct((2, NPAD, HC), jnp.float32),
            jax.ShapeDtypeStruct((2, DR, HC), jnp.float32),
        ],
        scratch_types=[
            pltpu.VMEM((CHUNK,), jnp.int32),
            pltpu.VMEM((CHUNK,), jnp.int32),
            pltpu.VMEM((CHUNK,), jnp.int32),
            pltpu.VMEM((CHUNK,), jnp.int32),
            pltpu.VMEM((CHUNK * H,), jnp.float32),
            pltpu.VMEM((CHUNK * H,), jnp.float32),
            pltpu.VMEM((CHUNK, HC), jnp.float32),
            pltpu.VMEM((CHUNK, HC), jnp.float32),
            pltpu.VMEM((CHUNK,), jnp.int32),
            pltpu.VMEM((CHUNK, HC), jnp.float32),
            pltpu.VMEM((CHUNK, HC), jnp.float32),
            pltpu.VMEM_SHARED((NPAD, HC), jnp.float32),
            pltpu.VMEM_SHARED((DR, HC), jnp.float32),
            pltpu.SemaphoreType.DMA,
            pltpu.SemaphoreType.DMA,
            pltpu.SemaphoreType.DMA,
        ],
    )
    zeros = jnp.zeros((CHUNK, HC), jnp.float32)
    return kfn(h, src, dst, tflat, zeros)


_R2 = 1000


def _postA_body(acc0_ref, acc1_ref, dp0_ref, dp1_ref, h_ref, asrc_ref,
                adst_ref, k_ref, bias_ref, b_ref, tmp_ref, stats_ref,
                s_ref, q_ref):
    i = pl.program_id(0)
    kval = k_ref[0, 0]
    m = acc0_ref[...] + acc1_ref[...]
    d4 = dp0_ref[...] + dp1_ref[...]
    z = asrc_ref[...] + adst_ref[...]
    z = jnp.maximum(z, 0.2 * z)
    t4 = jnp.exp(z - kval)
    bmat = b_ref[...]
    d128 = jnp.dot(d4 + t4, bmat, preferred_element_type=jnp.float32)
    t128 = jnp.dot(t4, bmat, preferred_element_type=jnp.float32)
    o = (m + t128 * h_ref[...]) / (d128 + 1e-16) + bias_ref[...]
    tmp_ref[...] = o
    ps = jnp.sum(o, axis=0, keepdims=True)
    pq = jnp.sum(o * o, axis=0, keepdims=True)
    prev_s = jnp.where(i == 0, jnp.zeros_like(ps), s_ref[...])
    prev_q = jnp.where(i == 0, jnp.zeros_like(pq), q_ref[...])
    s_ref[...] = prev_s + ps
    q_ref[...] = prev_q + pq
    stats_ref[0:1, :] = s_ref[...]
    stats_ref[1:2, :] = q_ref[...]


def _tc_postA(acc0, acc1, dp0, dp1, h, asrc, adst, kmat, bias, Bmat):
    return pl.pallas_call(
        _postA_body,
        grid=(N // _R2,),
        in_specs=[
            pl.BlockSpec((_R2, HC), lambda i: (i, 0)),
            pl.BlockSpec((_R2, HC), lambda i: (i, 0)),
            pl.BlockSpec((_R2, H), lambda i: (i, 0)),
            pl.BlockSpec((_R2, H), lambda i: (i, 0)),
            pl.BlockSpec((_R2, HC), lambda i: (i, 0)),
            pl.BlockSpec((_R2, H), lambda i: (i, 0)),
            pl.BlockSpec((_R2, H), lambda i: (i, 0)),
            pl.BlockSpec((8, HC), lambda i: (0, 0)),
            pl.BlockSpec((HC,), lambda i: (0,)),
            pl.BlockSpec((H, HC), lambda i: (0, 0)),
        ],
        out_specs=[
            pl.BlockSpec((_R2, HC), lambda i: (i, 0)),
            pl.BlockSpec((2, HC), lambda i: (0, 0)),
        ],
        out_shape=[
            jax.ShapeDtypeStruct((N, HC), jnp.float32),
            jax.ShapeDtypeStruct((2, HC), jnp.float32),
        ],
        scratch_shapes=[
            pltpu.VMEM((1, HC), jnp.float32),
            pltpu.VMEM((1, HC), jnp.float32),
        ],
    )(acc0, acc1, dp0, dp1, h, asrc, adst, kmat, bias, Bmat)


def _postB_body(tmp_ref, stats_ref, gamma_ref, beta_ref, x_ref, y_ref):
    s = stats_ref[0:1, :]
    q = stats_ref[1:2, :]
    mu = s * (1.0 / N)
    var = q * (1.0 / N) - mu * mu
    inv = jax.lax.rsqrt(var + 1e-5)
    yv = (tmp_ref[...] - mu) * inv * gamma_ref[...] + beta_ref[...]
    y_ref[...] = jnp.maximum(yv, 0.0) + x_ref[...]


def _tc_postB(tmp, stats, gamma, beta, x):
    return pl.pallas_call(
        _postB_body,
        grid=(N // _R2,),
        in_specs=[
            pl.BlockSpec((_R2, HC), lambda i: (i, 0)),
            pl.BlockSpec((2, HC), lambda i: (0, 0)),
            pl.BlockSpec((HC,), lambda i: (0,)),
            pl.BlockSpec((HC,), lambda i: (0,)),
            pl.BlockSpec((_R2, HC), lambda i: (i, 0)),
        ],
        out_specs=pl.BlockSpec((_R2, HC), lambda i: (i, 0)),
        out_shape=jax.ShapeDtypeStruct((N, HC), jnp.float32),
    )(tmp, stats, gamma, beta, x)


def kernel(x, edge_index, W, att_src, att_dst, bias, gamma, beta):
    rows = jnp.arange(HC)
    heads = rows // C
    A_src = jnp.where(heads[:, None] == jnp.arange(H)[None, :],
                      att_src.reshape(-1)[:, None], 0.0).astype(jnp.float32)
    A_dst = jnp.where(heads[:, None] == jnp.arange(H)[None, :],
                      att_dst.reshape(-1)[:, None], 0.0).astype(jnp.float32)
    Bmat = (jnp.arange(H)[:, None] == heads[None, :]).astype(jnp.float32)

    h, asrc, adst, kmat = _tc_pre(x, W, A_src, A_dst)

    npadedges = E_PAD - E
    src = jnp.concatenate([edge_index[0],
                           jnp.zeros((npadedges,), edge_index.dtype)])
    dst = jnp.concatenate([edge_index[1],
                           jnp.full((npadedges,), PADDST, edge_index.dtype)])
    k16 = kmat[0, :16]
    tpad = jnp.zeros((TP,), jnp.float32)
    tflat = _sc_logits(jnp.concatenate([asrc.reshape(-1), tpad]),
                       jnp.concatenate([adst.reshape(-1), tpad]),
                       src, dst, k16)
    acc, accd = _sc_edges(h, src, dst, tflat)

    dp0 = accd[0].reshape(NPAD, H)[:N]
    dp1 = accd[1].reshape(NPAD, H)[:N]
    tmp, stats = _tc_postA(acc[0, :N], acc[1, :N], dp0, dp1, h, asrc, adst,
                           kmat, bias, Bmat)
    return _tc_postB(tmp, stats, gamma, beta, x)
